# Initial kernel scaffold; baseline (speedup 1.0000x reference)
import jax
import jax.numpy as jnp
from jax.experimental import pallas as pl


def kernel(z, edge_index, edge_attr, W_self, W_nei, W_edge, gamma, beta):
    raise NotImplementedError("write your pallas kernel here")


# speedup vs baseline: 2.5368x; 2.5368x over previous
import functools

import jax
import jax.numpy as jnp
from jax import lax
from jax.experimental import pallas as pl
from jax.experimental.pallas import tpu as pltpu
from jax.experimental.pallas import tpu_sc as plsc

_N = 10000
_E = 320000
_D = 128
_ED = 16
_AUG = 144
_NC = 2
_NS = 16
_NW = _NC * _NS
_EPW = _E // _NW
_CH = 80
_NCHUNK = _EPW // _CH
_NP = 10240
_RPT = _NP // _NS
_ZB = 128


def _w_body(x_ref, wkT_ref, o_ref):
    wk = wkT_ref[...]
    e3 = (lax.broadcasted_iota(jnp.int32, (_D, _ED, _D), 0) ==
          lax.broadcasted_iota(jnp.int32, (_D, _ED, _D), 2))
    wb3 = jnp.where(e3, jnp.broadcast_to(wk[None], (_D, _ED, _D)), 0.0)
    wbig = wb3.reshape(_D * _ED, _D)
    s = jnp.dot(x_ref[...], wbig, preferred_element_type=jnp.float32)
    o_ref[...] = jnp.maximum(jax.nn.sigmoid(s), 1e-6)


def _edge_w(x, wkT):
    return pl.pallas_call(
        _w_body,
        out_shape=jax.ShapeDtypeStruct((_E // _D, _D), jnp.float32),
    )(x, wkT)


def _sc_body(z_hbm, src_hbm, dst_hbm, w_hbm, agg_hbm, deg_hbm,
             srcv, dstv, wv, gbuf, sbuf, degv, agg_sh, sem):
    cid = lax.axis_index("c")
    sid = lax.axis_index("s")
    wid = cid * _NS + sid
    base = wid * _EPW
    row0 = sid * _RPT

    zero16 = jnp.zeros((16,), jnp.float32)

    def zb_body(r, carry):
        for k in range(_D // 16):
            sbuf[r, pl.ds(k * 16, 16)] = zero16
            degv[r, pl.ds(k * 16, 16)] = zero16
        return carry

    lax.fori_loop(0, _CH, zb_body, 0)
    for j in range(_RPT // _CH):
        pltpu.sync_copy(sbuf, agg_sh.at[pl.ds(row0 + j * _CH, _CH)])

    plsc.subcore_barrier()

    ones16 = jnp.ones((16,), jnp.float32)

    def chunk(i, carry):
        off = base + i * _CH
        pltpu.sync_copy(src_hbm.at[pl.ds(off, _CH)], srcv)
        pltpu.sync_copy(dst_hbm.at[pl.ds(off, _CH)], dstv)
        pltpu.sync_copy(w_hbm.at[pl.ds(off, _CH)], wv)
        pltpu.async_copy(z_hbm.at[srcv], gbuf, sem).wait()

        def row(e, c2):
            eidx = jnp.full((16,), e, jnp.int32)
            ws = plsc.load_gather(wv, [eidx])
            for k in range(_D // 16):
                sbuf[e, pl.ds(k * 16, 16)] = gbuf[e, pl.ds(k * 16, 16)] * ws
            return c2

        lax.fori_loop(0, _CH, row, 0)
        for g in range(_CH // 16):
            dv = dstv[pl.ds(g * 16, 16)]
            plsc.addupdate_scatter(
                degv, [lax.shift_right_logical(dv, 7),
                       lax.bitwise_and(dv, 127)], ones16)
        pltpu.sync_copy(sbuf, agg_sh.at[dstv], add=True)
        return carry

    lax.fori_loop(0, _NCHUNK, chunk, 0)

    plsc.subcore_barrier()
    pltpu.sync_copy(agg_sh.at[pl.ds(row0, _RPT)],
                    agg_hbm.at[cid, pl.ds(row0, _RPT)])
    pltpu.sync_copy(degv, deg_hbm.at[cid, sid])


def _sc_scatter(z, src, dst, wflat):
    kern = pl.kernel(
        _sc_body,
        out_type=[
            jax.ShapeDtypeStruct((_NC, _NP, _D), jnp.float32),
            jax.ShapeDtypeStruct((_NC, _NS, _NP // _D, _D), jnp.float32),
        ],
        mesh=plsc.VectorSubcoreMesh(core_axis_name="c", subcore_axis_name="s"),
        compiler_params=pltpu.CompilerParams(needs_layout_passes=False),
        scratch_types=[
            pltpu.VMEM((_CH,), jnp.int32),
            pltpu.VMEM((_CH,), jnp.int32),
            pltpu.VMEM((_CH,), jnp.float32),
            pltpu.VMEM((_CH, _D), jnp.float32),
            pltpu.VMEM((_CH, _D), jnp.float32),
            pltpu.VMEM((_NP // _D, _D), jnp.float32),
            pltpu.VMEM_SHARED((_NP, _D), jnp.float32),
            pltpu.SemaphoreType.DMA,
        ],
    )
    return kern(z, src, dst, wflat)


def _degsum_body(d_ref, o_ref):
    o_ref[...] = jnp.sum(d_ref[...], axis=(0, 1))


def _degsum(deg_parts):
    return pl.pallas_call(
        _degsum_body,
        out_shape=jax.ShapeDtypeStruct((_NP // _D, _D), jnp.float32),
    )(deg_parts)


def _dense_body(z_ref, p0_ref, p1_ref, dc_ref, wsT_ref, wnT_ref, g_ref,
                b_ref, o_ref):
    z = z_ref[...]
    agg = p0_ref[0:_N, :] + p1_ref[0:_N, :]
    deg = jnp.maximum(dc_ref[...], 1.0)
    x2 = agg / deg
    h = jnp.dot(z, wsT_ref[...], preferred_element_type=jnp.float32)
    h = h + jnp.dot(x2, wnT_ref[...], preferred_element_type=jnp.float32)
    h = jnp.maximum(h, 0.0)
    mu = jnp.mean(h, axis=-1, keepdims=True)
    d = h - mu
    var = jnp.mean(d * d, axis=-1, keepdims=True)
    o_ref[...] = d * lax.rsqrt(var + 1e-5) * g_ref[...] + b_ref[...]


def _dense(z, p0, p1, dc, wsT, wnT, g, b):
    return pl.pallas_call(
        _dense_body,
        out_shape=jax.ShapeDtypeStruct((_N, _D), jnp.float32),
    )(z, p0, p1, dc, wsT, wnT, g, b)


def kernel(z, edge_index, edge_attr, W_self, W_nei, W_edge, gamma, beta):
    src = edge_index[0].astype(jnp.int32)
    dst = edge_index[1].astype(jnp.int32)
    x = edge_attr.reshape(_E // _D, _D * _ED)
    wflat = _edge_w(x, W_edge.reshape(_ED, 1)).reshape(_E)
    agg, deg_parts = _sc_scatter(z, src, dst, wflat)
    deg_col = _degsum(deg_parts).reshape(_NP, 1)[:_N]
    return _dense(z, agg[0], agg[1], deg_col, W_self.T, W_nei.T,
                  gamma.reshape(1, _D), beta.reshape(1, _D))


# speedup vs baseline: 5.4134x; 2.1340x over previous
import functools

import jax
import jax.numpy as jnp
from jax import lax
from jax.experimental import pallas as pl
from jax.experimental.pallas import tpu as pltpu
from jax.experimental.pallas import tpu_sc as plsc

_N = 10000
_E = 320000
_D = 128
_ED = 16
_AUG = 144
_NC = 2
_NS = 16
_NW = _NC * _NS
_EPW = _E // _NW
_CH = 80
_NCHUNK = _EPW // _CH
_NP = 10240
_RPT = _NP // _NS
_ZB = 128


def _w_body(x_ref, wkT_ref, o_ref):
    wk = wkT_ref[...]
    e3 = (lax.broadcasted_iota(jnp.int32, (_D, _ED, _D), 0) ==
          lax.broadcasted_iota(jnp.int32, (_D, _ED, _D), 2))
    wb3 = jnp.where(e3, jnp.broadcast_to(wk[None], (_D, _ED, _D)), 0.0)
    wbig = wb3.reshape(_D * _ED, _D)
    s = jnp.dot(x_ref[...], wbig, preferred_element_type=jnp.float32)
    o_ref[...] = jnp.maximum(jax.nn.sigmoid(s), 1e-6)


def _edge_w(x, wkT):
    return pl.pallas_call(
        _w_body,
        out_shape=jax.ShapeDtypeStruct((_E // _D, _D), jnp.float32),
    )(x, wkT)


def _sc_body(z_hbm, src_hbm, dst_hbm, w_hbm, agg_hbm, deg_hbm,
             srcv, dstv, wv, gbuf, degv, agg_sh,
             sem_g0, sem_g1, sem_i0, sem_i1):
    cid = lax.axis_index("c")
    sid = lax.axis_index("s")
    wid = cid * _NS + sid
    base = wid * _EPW
    row0 = sid * _RPT
    isem = [sem_i0, sem_i1]
    gsem = [sem_g0, sem_g1]

    zero16 = jnp.zeros((16,), jnp.float32)

    def zb_body(r, carry):
        for k in range(_D // 16):
            gbuf[0, r, pl.ds(k * 16, 16)] = zero16
            degv[r, pl.ds(k * 16, 16)] = zero16
        return carry

    lax.fori_loop(0, _CH, zb_body, 0)
    for j in range(_RPT // _CH):
        pltpu.sync_copy(gbuf.at[0], agg_sh.at[pl.ds(row0 + j * _CH, _CH)])

    plsc.subcore_barrier()

    ones16 = jnp.ones((16,), jnp.float32)

    def issue_idx(c, b):
        off = base + c * _CH
        pltpu.async_copy(src_hbm.at[pl.ds(off, _CH)], srcv.at[b], isem[b])
        pltpu.async_copy(dst_hbm.at[pl.ds(off, _CH)], dstv.at[b], isem[b])
        pltpu.async_copy(w_hbm.at[pl.ds(off, _CH)], wv.at[b], isem[b])

    def wait_idx(b):
        pltpu.make_async_copy(src_hbm.at[pl.ds(0, _CH)], srcv.at[b],
                              isem[b]).wait()
        pltpu.make_async_copy(dst_hbm.at[pl.ds(0, _CH)], dstv.at[b],
                              isem[b]).wait()
        pltpu.make_async_copy(w_hbm.at[pl.ds(0, _CH)], wv.at[b],
                              isem[b]).wait()

    def issue_gather(b):
        pltpu.async_copy(z_hbm.at[srcv.at[b]], gbuf.at[b], gsem[b])

    def wait_gather(b):
        pltpu.make_async_copy(z_hbm.at[srcv.at[b]], gbuf.at[b],
                              gsem[b]).wait()

    def scale_deg(b):
        def row(e, c2):
            eidx = jnp.full((16,), e, jnp.int32)
            ws = plsc.load_gather(wv.at[b], [eidx])
            for k in range(_D // 16):
                gbuf[b, e, pl.ds(k * 16, 16)] = (
                    gbuf[b, e, pl.ds(k * 16, 16)] * ws)
            return c2

        lax.fori_loop(0, _CH, row, 0)
        for g in range(_CH // 16):
            dv = dstv[b, pl.ds(g * 16, 16)]
            plsc.addupdate_scatter(
                degv, [lax.shift_right_logical(dv, 7),
                       lax.bitwise_and(dv, 127)], ones16)

    def step(c, b):
        nc = c + 1

        @pl.when(nc < _NCHUNK)
        def _():
            wait_idx(1 - b)

        wait_gather(b)

        @pl.when(nc < _NCHUNK)
        def _():
            issue_gather(1 - b)

        scale_deg(b)
        pltpu.sync_copy(gbuf.at[b], agg_sh.at[dstv.at[b]], add=True)

        @pl.when(c + 2 < _NCHUNK)
        def _():
            issue_idx(c + 2, b)

    pltpu.sync_copy(src_hbm.at[pl.ds(base, _CH)], srcv.at[0])
    pltpu.sync_copy(dst_hbm.at[pl.ds(base, _CH)], dstv.at[0])
    pltpu.sync_copy(w_hbm.at[pl.ds(base, _CH)], wv.at[0])
    issue_idx(1, 1)
    issue_gather(0)
    step(0, 0)

    def pair(t, carry):
        step(1 + 2 * t, 1)
        step(2 + 2 * t, 0)
        return carry

    lax.fori_loop(0, (_NCHUNK - 1) // 2, pair, 0)

    plsc.subcore_barrier()
    pltpu.sync_copy(agg_sh.at[pl.ds(row0, _RPT)],
                    agg_hbm.at[cid, pl.ds(row0, _RPT)])
    pltpu.sync_copy(degv, deg_hbm.at[cid, sid])


def _sc_scatter(z, src, dst, wflat):
    kern = pl.kernel(
        _sc_body,
        out_type=[
            jax.ShapeDtypeStruct((_NC, _NP, _D), jnp.float32),
            jax.ShapeDtypeStruct((_NC, _NS, _NP // _D, _D), jnp.float32),
        ],
        mesh=plsc.VectorSubcoreMesh(core_axis_name="c", subcore_axis_name="s"),
        compiler_params=pltpu.CompilerParams(needs_layout_passes=False),
        scratch_types=[
            pltpu.VMEM((2, _CH), jnp.int32),
            pltpu.VMEM((2, _CH), jnp.int32),
            pltpu.VMEM((2, _CH), jnp.float32),
            pltpu.VMEM((2, _CH, _D), jnp.float32),
            pltpu.VMEM((_NP // _D, _D), jnp.float32),
            pltpu.VMEM_SHARED((_NP, _D), jnp.float32),
            pltpu.SemaphoreType.DMA,
            pltpu.SemaphoreType.DMA,
            pltpu.SemaphoreType.DMA,
            pltpu.SemaphoreType.DMA,
        ],
    )
    return kern(z, src, dst, wflat)


def _degsum_body(d_ref, o_ref):
    o_ref[...] = jnp.sum(d_ref[...], axis=(0, 1))


def _degsum(deg_parts):
    return pl.pallas_call(
        _degsum_body,
        out_shape=jax.ShapeDtypeStruct((_NP // _D, _D), jnp.float32),
    )(deg_parts)


def _dense_body(z_ref, p0_ref, p1_ref, dc_ref, wsT_ref, wnT_ref, g_ref,
                b_ref, o_ref):
    z = z_ref[...]
    agg = p0_ref[0:_N, :] + p1_ref[0:_N, :]
    deg = jnp.maximum(dc_ref[...], 1.0)
    x2 = agg / deg
    h = jnp.dot(z, wsT_ref[...], preferred_element_type=jnp.float32)
    h = h + jnp.dot(x2, wnT_ref[...], preferred_element_type=jnp.float32)
    h = jnp.maximum(h, 0.0)
    mu = jnp.mean(h, axis=-1, keepdims=True)
    d = h - mu
    var = jnp.mean(d * d, axis=-1, keepdims=True)
    o_ref[...] = d * lax.rsqrt(var + 1e-5) * g_ref[...] + b_ref[...]


def _dense(z, p0, p1, dc, wsT, wnT, g, b):
    return pl.pallas_call(
        _dense_body,
        out_shape=jax.ShapeDtypeStruct((_N, _D), jnp.float32),
    )(z, p0, p1, dc, wsT, wnT, g, b)


def kernel(z, edge_index, edge_attr, W_self, W_nei, W_edge, gamma, beta):
    src = edge_index[0].astype(jnp.int32)
    dst = edge_index[1].astype(jnp.int32)
    x = edge_attr.reshape(_E // _D, _D * _ED)
    wflat = _edge_w(x, W_edge.reshape(_ED, 1)).reshape(_E)
    agg, deg_parts = _sc_scatter(z, src, dst, wflat)
    deg_col = _degsum(deg_parts).reshape(_NP, 1)[:_N]
    return _dense(z, agg[0], agg[1], deg_col, W_self.T, W_nei.T,
                  gamma.reshape(1, _D), beta.reshape(1, _D))


# speedup vs baseline: 5.8980x; 1.0895x over previous
import functools

import jax
import jax.numpy as jnp
from jax import lax
from jax.experimental import pallas as pl
from jax.experimental.pallas import tpu as pltpu
from jax.experimental.pallas import tpu_sc as plsc

_N = 10000
_E = 320000
_D = 128
_ED = 16
_AUG = 144
_NC = 2
_NS = 16
_NW = _NC * _NS
_EPW = _E // _NW
_CH = 80
_NCHUNK = _EPW // _CH
_NP = 10240
_RPT = _NP // _NS
_ZB = 128


def _w_body(x_ref, wkT_ref, o_ref):
    wk = wkT_ref[...]
    e3 = (lax.broadcasted_iota(jnp.int32, (_D, _ED, _D), 0) ==
          lax.broadcasted_iota(jnp.int32, (_D, _ED, _D), 2))
    wb3 = jnp.where(e3, jnp.broadcast_to(wk[None], (_D, _ED, _D)), 0.0)
    wbig = wb3.reshape(_D * _ED, _D)
    s = jnp.dot(x_ref[...], wbig, preferred_element_type=jnp.float32)
    o_ref[...] = jnp.maximum(jax.nn.sigmoid(s), 1e-6)


def _edge_w(x, wkT):
    return pl.pallas_call(
        _w_body,
        out_shape=jax.ShapeDtypeStruct((_E // _D, _D), jnp.float32),
    )(x, wkT)


_NG = _CH // 16


def _sc_body(z_hbm, src_hbm, dst_hbm, w_hbm, agg_hbm, deg_hbm,
             srcv, dstf, dstv0, dstv1, wv, gbuf, degv, agg_sh,
             sem_g0, sem_g1, sem_i0, sem_i1, sem_s0, sem_s1):
    dstv = [dstv0, dstv1]
    cid = lax.axis_index("c")
    sid = lax.axis_index("s")
    wid = cid * _NS + sid
    base = wid * _EPW
    row0 = sid * _RPT
    isem = [sem_i0, sem_i1]
    gsem = [sem_g0, sem_g1]
    ssem = [sem_s0, sem_s1]

    zero16 = jnp.zeros((16,), jnp.float32)

    def zb_body(r, carry):
        for k in range(_D // 16):
            gbuf[0, r, pl.ds(k * 16, 16)] = zero16
            degv[r, pl.ds(k * 16, 16)] = zero16
        return carry

    lax.fori_loop(0, _CH, zb_body, 0)
    for j in range(_RPT // _CH):
        pltpu.sync_copy(gbuf.at[0], agg_sh.at[pl.ds(row0 + j * _CH, _CH)])

    plsc.subcore_barrier()

    ones16 = jnp.ones((16,), jnp.float32)

    def issue_idx(c, b):
        off = base + c * _CH
        pltpu.async_copy(src_hbm.at[pl.ds(off, _CH)], srcv.at[b], isem[b])
        pltpu.async_copy(dst_hbm.at[pl.ds(off, _CH)], dstf.at[b], isem[b])
        pltpu.async_copy(w_hbm.at[pl.ds(off, _CH)], wv.at[b], isem[b])

    def wait_idx(b):
        pltpu.make_async_copy(src_hbm.at[pl.ds(0, _CH)], srcv.at[b],
                              isem[b]).wait()
        pltpu.make_async_copy(dst_hbm.at[pl.ds(0, _CH)], dstf.at[b],
                              isem[b]).wait()
        pltpu.make_async_copy(w_hbm.at[pl.ds(0, _CH)], wv.at[b],
                              isem[b]).wait()

    def issue_gather(b):
        pltpu.async_copy(z_hbm.at[srcv.at[b]], gbuf.at[b], gsem[b])

    def wait_gather(b):
        pltpu.make_async_copy(z_hbm.at[srcv.at[b]], gbuf.at[b],
                              gsem[b]).wait()

    def wait_scatter(b):
        for g in range(_NG):
            pltpu.make_async_copy(gbuf.at[b, pl.ds(g * 16, 16)],
                                  agg_sh.at[dstv[b][g]], ssem[b]).wait()

    def scale_deg_scatter(b):
        for g in range(_NG):
            wg = wv[b, pl.ds(g * 16, 16)]
            dv = dstf[b, pl.ds(g * 16, 16)]
            dstv[b][g][...] = dv
            for j in range(16):
                e = g * 16 + j
                ws = jnp.broadcast_to(wg[j], (16,))
                for k in range(_D // 16):
                    gbuf[b, e, pl.ds(k * 16, 16)] = (
                        gbuf[b, e, pl.ds(k * 16, 16)] * ws)
            plsc.addupdate_scatter(
                degv, [lax.shift_right_logical(dv, 7),
                       lax.bitwise_and(dv, 127)], ones16)
            pltpu.async_copy(gbuf.at[b, pl.ds(g * 16, 16)],
                             agg_sh.at[dstv[b][g]], ssem[b], add=True)

    def step(c, b, first=False):
        nc = c + 1

        @pl.when(nc < _NCHUNK)
        def _():
            wait_idx(1 - b)

        if not first:
            wait_scatter(1 - b)

        wait_gather(b)

        @pl.when(nc < _NCHUNK)
        def _():
            issue_gather(1 - b)

        scale_deg_scatter(b)

        @pl.when(c + 2 < _NCHUNK)
        def _():
            issue_idx(c + 2, b)

    pltpu.sync_copy(src_hbm.at[pl.ds(base, _CH)], srcv.at[0])
    pltpu.sync_copy(dst_hbm.at[pl.ds(base, _CH)], dstf.at[0])
    pltpu.sync_copy(w_hbm.at[pl.ds(base, _CH)], wv.at[0])
    issue_idx(1, 1)
    issue_gather(0)
    step(0, 0, first=True)

    def pair(t, carry):
        step(1 + 2 * t, 1)
        step(2 + 2 * t, 0)
        return carry

    lax.fori_loop(0, (_NCHUNK - 1) // 2, pair, 0)
    wait_scatter(0)

    plsc.subcore_barrier()
    pltpu.sync_copy(agg_sh.at[pl.ds(row0, _RPT)],
                    agg_hbm.at[cid, pl.ds(row0, _RPT)])
    pltpu.sync_copy(degv, deg_hbm.at[cid, sid])


def _sc_scatter(z, src, dst, wflat):
    kern = pl.kernel(
        _sc_body,
        out_type=[
            jax.ShapeDtypeStruct((_NC, _NP, _D), jnp.float32),
            jax.ShapeDtypeStruct((_NC, _NS, _NP // _D, _D), jnp.float32),
        ],
        mesh=plsc.VectorSubcoreMesh(core_axis_name="c", subcore_axis_name="s"),
        compiler_params=pltpu.CompilerParams(needs_layout_passes=False),
        scratch_types=[
            pltpu.VMEM((2, _CH), jnp.int32),
            pltpu.VMEM((2, _CH), jnp.int32),
            [pltpu.VMEM((16,), jnp.int32) for _ in range(_NG)],
            [pltpu.VMEM((16,), jnp.int32) for _ in range(_NG)],
            pltpu.VMEM((2, _CH), jnp.float32),
            pltpu.VMEM((2, _CH, _D), jnp.float32),
            pltpu.VMEM((_NP // _D, _D), jnp.float32),
            pltpu.VMEM_SHARED((_NP, _D), jnp.float32),
            pltpu.SemaphoreType.DMA,
            pltpu.SemaphoreType.DMA,
            pltpu.SemaphoreType.DMA,
            pltpu.SemaphoreType.DMA,
            pltpu.SemaphoreType.DMA,
            pltpu.SemaphoreType.DMA,
        ],
    )
    return kern(z, src, dst, wflat)


def _degsum_body(d_ref, o_ref):
    o_ref[...] = 1.0 / jnp.maximum(jnp.sum(d_ref[...], axis=(0, 1)), 1.0)


def _degsum(deg_parts):
    return pl.pallas_call(
        _degsum_body,
        out_shape=jax.ShapeDtypeStruct((_NP // _D, _D), jnp.float32),
    )(deg_parts)


def _dense_body(z_ref, a_ref, rc_ref, wsT_ref, wnT_ref, g_ref,
                b_ref, o_ref):
    z = z_ref[...]
    agg = a_ref[0, 0:_N, :] + a_ref[1, 0:_N, :]
    x2 = agg * rc_ref[...]
    h = jnp.dot(z, wsT_ref[...], preferred_element_type=jnp.float32)
    h = h + jnp.dot(x2, wnT_ref[...], preferred_element_type=jnp.float32)
    h = jnp.maximum(h, 0.0)
    mu = jnp.mean(h, axis=-1, keepdims=True)
    d = h - mu
    var = jnp.mean(d * d, axis=-1, keepdims=True)
    o_ref[...] = d * lax.rsqrt(var + 1e-5) * g_ref[...] + b_ref[...]


def _dense(z, agg, rcol, wsT, wnT, g, b):
    return pl.pallas_call(
        _dense_body,
        out_shape=jax.ShapeDtypeStruct((_N, _D), jnp.float32),
    )(z, agg, rcol, wsT, wnT, g, b)


def kernel(z, edge_index, edge_attr, W_self, W_nei, W_edge, gamma, beta):
    src = edge_index[0].astype(jnp.int32)
    dst = edge_index[1].astype(jnp.int32)
    x = edge_attr.reshape(_E // _D, _D * _ED)
    wflat = _edge_w(x, W_edge.reshape(_ED, 1)).reshape(_E)
    agg, deg_parts = _sc_scatter(z, src, dst, wflat)
    rcol = _degsum(deg_parts).reshape(_NP, 1)[:_N]
    return _dense(z, agg, rcol, W_self.T, W_nei.T,
                  gamma.reshape(1, _D), beta.reshape(1, _D))


# speedup vs baseline: 5.9619x; 1.0108x over previous
import jax
import jax.numpy as jnp
from jax import lax
from jax.experimental import pallas as pl
from jax.experimental.pallas import tpu as pltpu
from jax.experimental.pallas import tpu_sc as plsc

_N = 10000
_E = 320000
_D = 128
_ED = 16
_NC = 2
_NS = 16
_NW = _NC * _NS
_CH = 128
_NCH = _E // _CH
_CPT = _NCH // _NW
_NG = _CH // 16
_NP = 10240
_RPT = _NP // _NS


def _w_body(x_ref, wkT_ref, o_ref):
    wk = wkT_ref[...]
    e3 = (lax.broadcasted_iota(jnp.int32, (_D, _ED, _D), 0) ==
          lax.broadcasted_iota(jnp.int32, (_D, _ED, _D), 2))
    wb3 = jnp.where(e3, jnp.broadcast_to(wk[None], (_D, _ED, _D)), 0.0)
    wbig = wb3.reshape(_D * _ED, _D)
    s = jnp.dot(x_ref[...], wbig, preferred_element_type=jnp.float32)
    o_ref[...] = jnp.maximum(jax.nn.sigmoid(s), 1e-6)


def _edge_w(x, wkT):
    return pl.pallas_call(
        _w_body,
        out_shape=jax.ShapeDtypeStruct((_E // _D, _D), jnp.float32),
    )(x, wkT)


def _sc_body(z_hbm, ei_hbm, w_hbm, agg_hbm, deg_hbm,
             srcv, dstf, dstv0, dstv1, wv, gbuf, degv, agg_sh,
             sem_g0, sem_g1, sem_i0, sem_i1, sem_s0, sem_s1):
    dstv = [dstv0, dstv1]
    cid = lax.axis_index("c")
    sid = lax.axis_index("s")
    wid = cid * _NS + sid
    row0 = sid * _RPT
    nch = _CPT + (wid < 4).astype(jnp.int32)
    isem = [sem_i0, sem_i1]
    gsem = [sem_g0, sem_g1]
    ssem = [sem_s0, sem_s1]

    zero16 = jnp.zeros((16,), jnp.float32)

    def zb_body(r, carry):
        for k in range(_D // 16):
            gbuf[0, r, pl.ds(k * 16, 16)] = zero16
        return carry

    lax.fori_loop(0, _CH, zb_body, 0)

    def zd_body(r, carry):
        for k in range(_D // 16):
            degv[r, pl.ds(k * 16, 16)] = zero16
        return carry

    lax.fori_loop(0, _NP // _D, zd_body, 0)
    for j in range(_RPT // _CH):
        pltpu.sync_copy(gbuf.at[0], agg_sh.at[pl.ds(row0 + j * _CH, _CH)])

    plsc.subcore_barrier()

    ones16 = jnp.ones((16,), jnp.float32)

    def soff(k):
        return (wid + _NW * k) * _CH

    def issue_idx(k, b):
        pltpu.async_copy(ei_hbm.at[pl.ds(soff(k), _CH)], srcv.at[b], isem[b])
        pltpu.async_copy(ei_hbm.at[pl.ds(_E + soff(k), _CH)], dstf.at[b],
                         isem[b])
        pltpu.async_copy(w_hbm.at[pl.ds(soff(k), _CH)], wv.at[b], isem[b])

    def wait_idx(b):
        pltpu.make_async_copy(ei_hbm.at[pl.ds(0, _CH)], srcv.at[b],
                              isem[b]).wait()
        pltpu.make_async_copy(ei_hbm.at[pl.ds(0, _CH)], dstf.at[b],
                              isem[b]).wait()
        pltpu.make_async_copy(w_hbm.at[pl.ds(0, _CH)], wv.at[b],
                              isem[b]).wait()

    def issue_gather(b):
        pltpu.async_copy(z_hbm.at[srcv.at[b]], gbuf.at[b], gsem[b])

    def wait_gather(b):
        pltpu.make_async_copy(z_hbm.at[srcv.at[b]], gbuf.at[b],
                              gsem[b]).wait()

    def wait_scatter(b):
        for g in range(_NG):
            pltpu.make_async_copy(gbuf.at[b, pl.ds(g * 16, 16)],
                                  agg_sh.at[dstv[b][g]], ssem[b]).wait()

    def scale_deg_scatter(b):
        for g in range(_NG):
            wg = wv[b, pl.ds(g * 16, 16)]
            dv = dstf[b, pl.ds(g * 16, 16)]
            dstv[b][g][...] = dv
            for j in range(16):
                e = g * 16 + j
                ws = jnp.broadcast_to(wg[j], (16,))
                for kk in range(_D // 16):
                    gbuf[b, e, pl.ds(kk * 16, 16)] = (
                        gbuf[b, e, pl.ds(kk * 16, 16)] * ws)
            plsc.addupdate_scatter(
                degv, [lax.shift_right_logical(dv, 7),
                       lax.bitwise_and(dv, 127)], ones16)
            pltpu.async_copy(gbuf.at[b, pl.ds(g * 16, 16)],
                             agg_sh.at[dstv[b][g]], ssem[b], add=True)

    def step(k, b):
        @pl.when(k < nch)
        def _():
            @pl.when(k + 1 < nch)
            def _():
                wait_idx(1 - b)

            @pl.when(k >= 1)
            def _():
                wait_scatter(1 - b)

            wait_gather(b)

            @pl.when(k + 1 < nch)
            def _():
                issue_gather(1 - b)

            scale_deg_scatter(b)

            @pl.when(k + 2 < nch)
            def _():
                issue_idx(k + 2, b)

    pltpu.sync_copy(ei_hbm.at[pl.ds(soff(0), _CH)], srcv.at[0])
    pltpu.sync_copy(ei_hbm.at[pl.ds(_E + soff(0), _CH)], dstf.at[0])
    pltpu.sync_copy(w_hbm.at[pl.ds(soff(0), _CH)], wv.at[0])
    issue_idx(1, 1)
    issue_gather(0)

    def pair(t, carry):
        step(2 * t, 0)
        step(2 * t + 1, 1)
        return carry

    lax.fori_loop(0, (_CPT + 2) // 2, pair, 0)

    @pl.when(wid < 4)
    def _():
        wait_scatter(_CPT % 2)
    @pl.when(wid >= 4)
    def _():
        wait_scatter(1 - _CPT % 2)

    plsc.subcore_barrier()
    pltpu.sync_copy(agg_sh.at[pl.ds(row0, _RPT)],
                    agg_hbm.at[cid, pl.ds(row0, _RPT)])
    pltpu.sync_copy(degv, deg_hbm.at[cid, sid])


def _sc_scatter(z, eif, wflat):
    kern = pl.kernel(
        _sc_body,
        out_type=[
            jax.ShapeDtypeStruct((_NC, _NP, _D), jnp.float32),
            jax.ShapeDtypeStruct((_NC, _NS, _NP // _D, _D), jnp.float32),
        ],
        mesh=plsc.VectorSubcoreMesh(core_axis_name="c", subcore_axis_name="s"),
        compiler_params=pltpu.CompilerParams(needs_layout_passes=False),
        scratch_types=[
            pltpu.VMEM((2, _CH), jnp.int32),
            pltpu.VMEM((2, _CH), jnp.int32),
            [pltpu.VMEM((16,), jnp.int32) for _ in range(_NG)],
            [pltpu.VMEM((16,), jnp.int32) for _ in range(_NG)],
            pltpu.VMEM((2, _CH), jnp.float32),
            pltpu.VMEM((2, _CH, _D), jnp.float32),
            pltpu.VMEM((_NP // _D, _D), jnp.float32),
            pltpu.VMEM_SHARED((_NP, _D), jnp.float32),
            pltpu.SemaphoreType.DMA,
            pltpu.SemaphoreType.DMA,
            pltpu.SemaphoreType.DMA,
            pltpu.SemaphoreType.DMA,
            pltpu.SemaphoreType.DMA,
            pltpu.SemaphoreType.DMA,
        ],
    )
    return kern(z, eif, wflat)


def _degsum_body(d_ref, o_ref):
    o_ref[...] = 1.0 / jnp.maximum(jnp.sum(d_ref[...], axis=(0, 1)), 1.0)


def _degsum(deg_parts):
    return pl.pallas_call(
        _degsum_body,
        out_shape=jax.ShapeDtypeStruct((_NP // _D, _D), jnp.float32),
    )(deg_parts)


def _dense_body(z_ref, a_ref, rc_ref, wsT_ref, wnT_ref, g_ref,
                b_ref, o_ref):
    z = z_ref[...]
    agg = a_ref[0, 0:_N, :] + a_ref[1, 0:_N, :]
    x2 = agg * rc_ref[...]
    h = jnp.dot(z, wsT_ref[...], preferred_element_type=jnp.float32)
    h = h + jnp.dot(x2, wnT_ref[...], preferred_element_type=jnp.float32)
    h = jnp.maximum(h, 0.0)
    mu = jnp.mean(h, axis=-1, keepdims=True)
    d = h - mu
    var = jnp.mean(d * d, axis=-1, keepdims=True)
    o_ref[...] = d * lax.rsqrt(var + 1e-5) * g_ref[...] + b_ref[...]


def _dense(z, agg, rcol, wsT, wnT, g, b):
    return pl.pallas_call(
        _dense_body,
        out_shape=jax.ShapeDtypeStruct((_N, _D), jnp.float32),
    )(z, agg, rcol, wsT, wnT, g, b)


def kernel(z, edge_index, edge_attr, W_self, W_nei, W_edge, gamma, beta):
    eif = edge_index.astype(jnp.int32).reshape(2 * _E)
    x = edge_attr.reshape(_E // _D, _D * _ED)
    wflat = _edge_w(x, W_edge.reshape(_ED, 1)).reshape(_E)
    agg, deg_parts = _sc_scatter(z, eif, wflat)
    rcol = _degsum(deg_parts).reshape(_NP, 1)[:_N]
    return _dense(z, agg, rcol, W_self.T, W_nei.T,
                  gamma.reshape(1, _D), beta.reshape(1, _D))


# speedup vs baseline: 6.1031x; 1.0237x over previous
import jax
import jax.numpy as jnp
from jax import lax
from jax.experimental import pallas as pl
from jax.experimental.pallas import tpu as pltpu
from jax.experimental.pallas import tpu_sc as plsc

_N = 10000
_E = 320000
_D = 128
_ED = 16
_NC = 2
_NS = 16
_NW = _NC * _NS
_CH = 128
_NCH = _E // _CH
_CPT = _NCH // _NW
_NG = _CH // 16
_NP = 10240
_RPT = _NP // _NS


def _w_body(x_ref, wkT_ref, o_ref):
    wk = wkT_ref[...]
    e3 = (lax.broadcasted_iota(jnp.int32, (_D, _ED, _D), 0) ==
          lax.broadcasted_iota(jnp.int32, (_D, _ED, _D), 2))
    wb3 = jnp.where(e3, jnp.broadcast_to(wk[None], (_D, _ED, _D)), 0.0)
    wbig = wb3.reshape(_D * _ED, _D)
    s = jnp.dot(x_ref[...], wbig, preferred_element_type=jnp.float32)
    o_ref[...] = jnp.maximum(jax.nn.sigmoid(s), 1e-6)


def _edge_w(x, wkT):
    return pl.pallas_call(
        _w_body,
        out_shape=jax.ShapeDtypeStruct((_E // _D, _D), jnp.float32),
    )(x, wkT)


def _sc_body(z_hbm, ei_hbm, w_hbm, agg_hbm, deg_hbm,
             srcv, dstf, dstv0, dstv1, wv, gbuf, degv, agg_sh,
             sem_g0, sem_g1, sem_i0, sem_i1, sem_s0, sem_s1):
    dstv = [dstv0, dstv1]
    cid = lax.axis_index("c")
    sid = lax.axis_index("s")
    wid = cid * _NS + sid
    row0 = sid * _RPT
    nch = _CPT + (wid < 4).astype(jnp.int32)
    isem = [sem_i0, sem_i1]
    gsem = [sem_g0, sem_g1]
    ssem = [sem_s0, sem_s1]

    zero16 = jnp.zeros((16,), jnp.float32)

    def zb_body(r, carry):
        for k in range(_D // 16):
            gbuf[0, r, pl.ds(k * 16, 16)] = zero16
        return carry

    lax.fori_loop(0, _CH, zb_body, 0)

    def zd_body(r, carry):
        for k in range(_D // 16):
            degv[r, pl.ds(k * 16, 16)] = zero16
        return carry

    lax.fori_loop(0, _NP // _D, zd_body, 0)
    for j in range(_RPT // _CH):
        pltpu.sync_copy(gbuf.at[0], agg_sh.at[pl.ds(row0 + j * _CH, _CH)])

    plsc.subcore_barrier()

    ones16 = jnp.ones((16,), jnp.float32)

    def soff(k):
        return (wid + _NW * k) * _CH

    def issue_idx(k, b):
        pltpu.async_copy(ei_hbm.at[0, pl.ds(soff(k), _CH)], srcv.at[b],
                         isem[b])
        pltpu.async_copy(ei_hbm.at[1, pl.ds(soff(k), _CH)], dstf.at[b],
                         isem[b])
        pltpu.async_copy(w_hbm.at[pl.ds(soff(k), _CH)], wv.at[b], isem[b])

    def wait_idx(b):
        pltpu.make_async_copy(ei_hbm.at[0, pl.ds(0, _CH)], srcv.at[b],
                              isem[b]).wait()
        pltpu.make_async_copy(ei_hbm.at[1, pl.ds(0, _CH)], dstf.at[b],
                              isem[b]).wait()
        pltpu.make_async_copy(w_hbm.at[pl.ds(0, _CH)], wv.at[b],
                              isem[b]).wait()

    def issue_gather(b):
        pltpu.async_copy(z_hbm.at[srcv.at[b]], gbuf.at[b], gsem[b])

    def wait_gather(b):
        pltpu.make_async_copy(z_hbm.at[srcv.at[b]], gbuf.at[b],
                              gsem[b]).wait()

    def wait_scatter(b):
        for g in range(_NG):
            pltpu.make_async_copy(gbuf.at[b, pl.ds(g * 16, 16)],
                                  agg_sh.at[dstv[b][g]], ssem[b]).wait()

    def scale_deg_scatter(b):
        for g in range(_NG):
            wg = wv[b, pl.ds(g * 16, 16)]
            dv = dstf[b, pl.ds(g * 16, 16)]
            dstv[b][g][...] = dv
            for j in range(16):
                e = g * 16 + j
                ws = jnp.broadcast_to(wg[j], (16,))
                for kk in range(_D // 16):
                    gbuf[b, e, pl.ds(kk * 16, 16)] = (
                        gbuf[b, e, pl.ds(kk * 16, 16)] * ws)
            plsc.addupdate_scatter(
                degv, [lax.shift_right_logical(dv, 7),
                       lax.bitwise_and(dv, 127)], ones16)
            pltpu.async_copy(gbuf.at[b, pl.ds(g * 16, 16)],
                             agg_sh.at[dstv[b][g]], ssem[b], add=True)

    def step(k, b):
        @pl.when(k < nch)
        def _():
            @pl.when(k + 1 < nch)
            def _():
                wait_idx(1 - b)

            @pl.when(k >= 1)
            def _():
                wait_scatter(1 - b)

            wait_gather(b)

            @pl.when(k + 1 < nch)
            def _():
                issue_gather(1 - b)

            scale_deg_scatter(b)

            @pl.when(k + 2 < nch)
            def _():
                issue_idx(k + 2, b)

    pltpu.sync_copy(ei_hbm.at[0, pl.ds(soff(0), _CH)], srcv.at[0])
    pltpu.sync_copy(ei_hbm.at[1, pl.ds(soff(0), _CH)], dstf.at[0])
    pltpu.sync_copy(w_hbm.at[pl.ds(soff(0), _CH)], wv.at[0])
    issue_idx(1, 1)
    issue_gather(0)

    def pair(t, carry):
        step(2 * t, 0)
        step(2 * t + 1, 1)
        return carry

    lax.fori_loop(0, (_CPT + 2) // 2, pair, 0)

    @pl.when(wid < 4)
    def _():
        wait_scatter(_CPT % 2)
    @pl.when(wid >= 4)
    def _():
        wait_scatter(1 - _CPT % 2)

    plsc.subcore_barrier()
    pltpu.sync_copy(agg_sh.at[pl.ds(row0, _RPT)],
                    agg_hbm.at[cid, pl.ds(row0, _RPT)])
    pltpu.sync_copy(degv, deg_hbm.at[cid, sid])


def _sc_scatter(z, eif, wflat):
    kern = pl.kernel(
        _sc_body,
        out_type=[
            jax.ShapeDtypeStruct((_NC, _NP, _D), jnp.float32),
            jax.ShapeDtypeStruct((_NC, _NS, _NP // _D, _D), jnp.float32),
        ],
        mesh=plsc.VectorSubcoreMesh(core_axis_name="c", subcore_axis_name="s"),
        compiler_params=pltpu.CompilerParams(needs_layout_passes=False),
        scratch_types=[
            pltpu.VMEM((2, _CH), jnp.int32),
            pltpu.VMEM((2, _CH), jnp.int32),
            [pltpu.VMEM((16,), jnp.int32) for _ in range(_NG)],
            [pltpu.VMEM((16,), jnp.int32) for _ in range(_NG)],
            pltpu.VMEM((2, _CH), jnp.float32),
            pltpu.VMEM((2, _CH, _D), jnp.float32),
            pltpu.VMEM((_NP // _D, _D), jnp.float32),
            pltpu.VMEM_SHARED((_NP, _D), jnp.float32),
            pltpu.SemaphoreType.DMA,
            pltpu.SemaphoreType.DMA,
            pltpu.SemaphoreType.DMA,
            pltpu.SemaphoreType.DMA,
            pltpu.SemaphoreType.DMA,
            pltpu.SemaphoreType.DMA,
        ],
    )
    return kern(z, eif, wflat)


_NB = _NP // _D


def _dense_body(z_ref, a_ref, dp_ref, wsT_ref, wnT_ref, g_ref,
                b_ref, o_ref):
    z = z_ref[...]
    af = a_ref[0] + a_ref[1]
    rden = 1.0 / jnp.maximum(jnp.sum(dp_ref[...], axis=(0, 1)), 1.0)
    eye3 = (lax.broadcasted_iota(jnp.int32, (_NB, _D, _D), 1) ==
            lax.broadcasted_iota(jnp.int32, (_NB, _D, _D), 2))
    diag3 = jnp.where(eye3,
                      jnp.broadcast_to(rden[:, None, :], (_NB, _D, _D)), 0.0)
    a3 = af.reshape(_NB, _D, _D)
    x23 = lax.dot_general(diag3, a3, (((2,), (1,)), ((0,), (0,))),
                          preferred_element_type=jnp.float32)
    x2 = x23.reshape(_NP, _D)[0:_N]
    h = jnp.dot(z, wsT_ref[...], preferred_element_type=jnp.float32)
    h = h + jnp.dot(x2, wnT_ref[...], preferred_element_type=jnp.float32)
    h = jnp.maximum(h, 0.0)
    mu = jnp.mean(h, axis=-1, keepdims=True)
    d = h - mu
    var = jnp.mean(d * d, axis=-1, keepdims=True)
    o_ref[...] = d * lax.rsqrt(var + 1e-5) * g_ref[...] + b_ref[...]


def _dense(z, agg, deg_parts, wsT, wnT, g, b):
    return pl.pallas_call(
        _dense_body,
        out_shape=jax.ShapeDtypeStruct((_N, _D), jnp.float32),
    )(z, agg, deg_parts, wsT, wnT, g, b)


def kernel(z, edge_index, edge_attr, W_self, W_nei, W_edge, gamma, beta):
    ei = edge_index.astype(jnp.int32)
    x = edge_attr.reshape(_E // _D, _D * _ED)
    wflat = _edge_w(x, W_edge.reshape(_ED, 1)).reshape(_E)
    agg, deg_parts = _sc_scatter(z, ei, wflat)
    return _dense(z, agg, deg_parts, W_self.T, W_nei.T,
                  gamma.reshape(1, _D), beta.reshape(1, _D))


# speedup vs baseline: 6.1248x; 1.0036x over previous
import jax
import jax.numpy as jnp
from jax import lax
from jax.experimental import pallas as pl
from jax.experimental.pallas import tpu as pltpu
from jax.experimental.pallas import tpu_sc as plsc

_N = 10000
_E = 320000
_D = 128
_ED = 16
_NC = 2
_NS = 16
_NW = _NC * _NS
_CH = 128
_NCH = _E // _CH
_CPT = _NCH // _NW
_NG = _CH // 16
_NP = 10240
_RPT = _NP // _NS


_WR = 2560


def _w_body(x_ref, wkT_ref, o_ref):
    wk = wkT_ref[...]
    e3 = (lax.broadcasted_iota(jnp.int32, (_D, _ED, _D), 0) ==
          lax.broadcasted_iota(jnp.int32, (_D, _ED, _D), 2))
    wb3 = jnp.where(e3, jnp.broadcast_to(wk[None], (_D, _ED, _D)), 0.0)
    wbig = wb3.reshape(_D * _ED, _D)
    s = jnp.dot(x_ref[...], wbig, preferred_element_type=jnp.float32)
    s = jnp.concatenate(
        [s, jnp.zeros((_WR - _E // _D, _D), jnp.float32)], axis=0)
    o_ref[...] = jnp.maximum(jax.nn.sigmoid(s), 1e-6)


def _edge_w(x, wkT):
    return pl.pallas_call(
        _w_body,
        out_shape=jax.ShapeDtypeStruct((_WR, _D), jnp.float32),
    )(x, wkT)


def _sc_body(z_hbm, ei_hbm, w_hbm, agg_hbm, deg_hbm,
             srcv, dstf, dstv0, dstv1, wv, gbuf, degv, agg_sh,
             sem_g0, sem_g1, sem_i0, sem_i1, sem_s0, sem_s1):
    dstv = [dstv0, dstv1]
    cid = lax.axis_index("c")
    sid = lax.axis_index("s")
    wid = cid * _NS + sid
    row0 = sid * _RPT
    nch = _CPT + (wid < 4).astype(jnp.int32)
    isem = [sem_i0, sem_i1]
    gsem = [sem_g0, sem_g1]
    ssem = [sem_s0, sem_s1]

    zero16 = jnp.zeros((16,), jnp.float32)

    def zb_body(r, carry):
        for k in range(_D // 16):
            gbuf[0, r, pl.ds(k * 16, 16)] = zero16
        return carry

    lax.fori_loop(0, _CH, zb_body, 0)

    def zd_body(r, carry):
        for k in range(_D // 16):
            degv[r, pl.ds(k * 16, 16)] = zero16
        return carry

    lax.fori_loop(0, _NP // _D, zd_body, 0)
    for j in range(_RPT // _CH):
        pltpu.sync_copy(gbuf.at[0], agg_sh.at[pl.ds(row0 + j * _CH, _CH)])

    plsc.subcore_barrier()

    ones16 = jnp.ones((16,), jnp.float32)

    def soff(k):
        return (wid + _NW * k) * _CH

    def issue_idx(k, b):
        pltpu.async_copy(ei_hbm.at[0, pl.ds(soff(k), _CH)], srcv.at[b],
                         isem[b])
        pltpu.async_copy(ei_hbm.at[1, pl.ds(soff(k), _CH)], dstf.at[b],
                         isem[b])
        pltpu.async_copy(w_hbm.at[pl.ds(soff(k), _CH)], wv.at[b], isem[b])

    def wait_idx(b):
        pltpu.make_async_copy(ei_hbm.at[0, pl.ds(0, _CH)], srcv.at[b],
                              isem[b]).wait()
        pltpu.make_async_copy(ei_hbm.at[1, pl.ds(0, _CH)], dstf.at[b],
                              isem[b]).wait()
        pltpu.make_async_copy(w_hbm.at[pl.ds(0, _CH)], wv.at[b],
                              isem[b]).wait()

    def issue_gather(b):
        pltpu.async_copy(z_hbm.at[srcv.at[b]], gbuf.at[b], gsem[b])

    def wait_gather(b):
        pltpu.make_async_copy(z_hbm.at[srcv.at[b]], gbuf.at[b],
                              gsem[b]).wait()

    def wait_scatter(b):
        for g in range(_NG):
            pltpu.make_async_copy(gbuf.at[b, pl.ds(g * 16, 16)],
                                  agg_sh.at[dstv[b][g]], ssem[b]).wait()

    def scale_deg_scatter(b):
        for g in range(_NG):
            wg = wv[b, pl.ds(g * 16, 16)]
            dv = dstf[b, pl.ds(g * 16, 16)]
            dstv[b][g][...] = dv
            for j in range(16):
                e = g * 16 + j
                ws = jnp.broadcast_to(wg[j], (16,))
                for kk in range(_D // 16):
                    gbuf[b, e, pl.ds(kk * 16, 16)] = (
                        gbuf[b, e, pl.ds(kk * 16, 16)] * ws)
            plsc.addupdate_scatter(
                degv, [lax.shift_right_logical(dv, 7),
                       lax.bitwise_and(dv, 127)], ones16)
            pltpu.async_copy(gbuf.at[b, pl.ds(g * 16, 16)],
                             agg_sh.at[dstv[b][g]], ssem[b], add=True)

    def step(k, b):
        @pl.when(k < nch)
        def _():
            @pl.when(k + 1 < nch)
            def _():
                wait_idx(1 - b)

            @pl.when(k >= 1)
            def _():
                wait_scatter(1 - b)

            wait_gather(b)

            @pl.when(k + 1 < nch)
            def _():
                issue_gather(1 - b)

            scale_deg_scatter(b)

            @pl.when(k + 2 < nch)
            def _():
                issue_idx(k + 2, b)

    pltpu.sync_copy(ei_hbm.at[0, pl.ds(soff(0), _CH)], srcv.at[0])
    pltpu.sync_copy(ei_hbm.at[1, pl.ds(soff(0), _CH)], dstf.at[0])
    pltpu.sync_copy(w_hbm.at[pl.ds(soff(0), _CH)], wv.at[0])
    issue_idx(1, 1)
    issue_gather(0)

    def pair(t, carry):
        step(2 * t, 0)
        step(2 * t + 1, 1)
        return carry

    lax.fori_loop(0, (_CPT + 2) // 2, pair, 0)

    @pl.when(wid < 4)
    def _():
        wait_scatter(_CPT % 2)
    @pl.when(wid >= 4)
    def _():
        wait_scatter(1 - _CPT % 2)

    plsc.subcore_barrier()
    pltpu.sync_copy(agg_sh.at[pl.ds(row0, _RPT)],
                    agg_hbm.at[cid, pl.ds(row0, _RPT)])
    pltpu.sync_copy(degv, deg_hbm.at[cid, sid])


def _sc_scatter(z, eif, wflat):
    kern = pl.kernel(
        _sc_body,
        out_type=[
            jax.ShapeDtypeStruct((_NC, _NP, _D), jnp.float32),
            jax.ShapeDtypeStruct((_NC, _NS, _NP // _D, _D), jnp.float32),
        ],
        mesh=plsc.VectorSubcoreMesh(core_axis_name="c", subcore_axis_name="s"),
        compiler_params=pltpu.CompilerParams(needs_layout_passes=False),
        scratch_types=[
            pltpu.VMEM((2, _CH), jnp.int32),
            pltpu.VMEM((2, _CH), jnp.int32),
            [pltpu.VMEM((16,), jnp.int32) for _ in range(_NG)],
            [pltpu.VMEM((16,), jnp.int32) for _ in range(_NG)],
            pltpu.VMEM((2, _CH), jnp.float32),
            pltpu.VMEM((2, _CH, _D), jnp.float32),
            pltpu.VMEM((_NP // _D, _D), jnp.float32),
            pltpu.VMEM_SHARED((_NP, _D), jnp.float32),
            pltpu.SemaphoreType.DMA,
            pltpu.SemaphoreType.DMA,
            pltpu.SemaphoreType.DMA,
            pltpu.SemaphoreType.DMA,
            pltpu.SemaphoreType.DMA,
            pltpu.SemaphoreType.DMA,
        ],
    )
    return kern(z, eif, wflat)


_NB = _NP // _D


def _dense_body(z_ref, a_ref, dp_ref, wsT_ref, wnT_ref, g_ref,
                b_ref, o_ref):
    z = z_ref[...]
    af = a_ref[0] + a_ref[1]
    rden = 1.0 / jnp.maximum(jnp.sum(dp_ref[...], axis=(0, 1)), 1.0)
    eye3 = (lax.broadcasted_iota(jnp.int32, (_NB, _D, _D), 1) ==
            lax.broadcasted_iota(jnp.int32, (_NB, _D, _D), 2))
    diag3 = jnp.where(eye3,
                      jnp.broadcast_to(rden[:, None, :], (_NB, _D, _D)), 0.0)
    a3 = af.reshape(_NB, _D, _D)
    x23 = lax.dot_general(diag3, a3, (((2,), (1,)), ((0,), (0,))),
                          preferred_element_type=jnp.float32)
    x2 = x23.reshape(_NP, _D)[0:_N]
    h = jnp.dot(z, wsT_ref[...], preferred_element_type=jnp.float32)
    h = h + jnp.dot(x2, wnT_ref[...], preferred_element_type=jnp.float32)
    h = jnp.maximum(h, 0.0)
    mu = jnp.mean(h, axis=-1, keepdims=True)
    d = h - mu
    var = jnp.mean(d * d, axis=-1, keepdims=True)
    o_ref[...] = d * lax.rsqrt(var + 1e-5) * g_ref[...] + b_ref[...]


def _dense(z, agg, deg_parts, wsT, wnT, g, b):
    return pl.pallas_call(
        _dense_body,
        out_shape=jax.ShapeDtypeStruct((_N, _D), jnp.float32),
    )(z, agg, deg_parts, wsT, wnT, g, b)


def kernel(z, edge_index, edge_attr, W_self, W_nei, W_edge, gamma, beta):
    ei = edge_index.astype(jnp.int32)
    x = edge_attr.reshape(_E // _D, _D * _ED)
    wflat = _edge_w(x, W_edge.reshape(_ED, 1)).reshape(_WR * _D)
    agg, deg_parts = _sc_scatter(z, ei, wflat)
    return _dense(z, agg, deg_parts, W_self.T, W_nei.T,
                  gamma.reshape(1, _D), beta.reshape(1, _D))
